# baseline (speedup 1.0000x reference)
import functools

import jax
import jax.numpy as jnp
from jax import lax
from jax.experimental import pallas as pl
from jax.experimental.pallas import tpu as pltpu
from jax.experimental.pallas import tpu_sc as plsc

_N = 50000
_E = 800000
_G = 256
_D = 64
_NP = 50176
_BN = 1024
_NBLK = _NP // _BN

_NW = 32
_EPW = _E // _NW
_C = 128
_NFULL = _EPW // _C
_TAIL = _EPW - _NFULL * _C
_R = 64


def _prep_edge_body(edge_ref, we_ref, be_ref, ge_ref, bee_ref, etab_ref):
    er = jnp.dot(edge_ref[...], we_ref[...],
                 preferred_element_type=jnp.float32) + be_ref[...]
    er = er * jax.nn.sigmoid(er)
    m = jnp.mean(er, axis=-1, keepdims=True)
    v = jnp.mean((er - m) ** 2, axis=-1, keepdims=True)
    etab_ref[...] = (er - m) / jnp.sqrt(v + 1e-5) * ge_ref[...] + bee_ref[...]


def _prep_edge(edge_table, w_e, b_e, g_e, beta_e):
    return pl.pallas_call(
        _prep_edge_body,
        out_shape=jax.ShapeDtypeStruct((8, 128), jnp.float32),
    )(edge_table, w_e, b_e, g_e, beta_e)


def _prep_node_body(batch_ref, t_ref, atom_ref, nc_ref, time_ref,
                    wh_ref, bh_ref, ta_ref, tg_ref):
    gio = lax.broadcasted_iota(jnp.int32, (_G, _BN), 0)

    def step(i, acc):
        row = batch_ref[pl.ds(i, 1), :]
        cmp = (row == gio).astype(jnp.float32)
        return acc + jnp.sum(cmp, axis=1, keepdims=True)

    counts = lax.fori_loop(0, _NBLK, step,
                           jnp.zeros((_G, 1), jnp.float32))
    counts = jnp.clip(counts.astype(jnp.int32), 0, 1023)

    vio = lax.broadcasted_iota(jnp.int32, (_G, 1024), 1)
    nc_oh = (counts == vio).astype(jnp.float32)
    nc_g = jnp.dot(nc_oh, nc_ref[...],
                   preferred_element_type=jnp.float32)
    t_oh = (t_ref[...] == vio).astype(jnp.float32)
    t_g = jnp.dot(t_oh, time_ref[...],
                  preferred_element_type=jnp.float32)

    wh = wh_ref[...]
    tg_ref[...] = (
        jnp.dot(nc_g, wh[64:128, :], preferred_element_type=jnp.float32)
        + jnp.dot(t_g, wh[128:192, :], preferred_element_type=jnp.float32)
        + bh_ref[...])
    ta_ref[...] = jnp.dot(atom_ref[...], wh[0:64, :],
                          preferred_element_type=jnp.float32)


def _prep_node(batch2d, t_col, atom_table, nc_table, time_pad, w_h0, b_h0):
    return pl.pallas_call(
        _prep_node_body,
        out_shape=[
            jax.ShapeDtypeStruct((128, 256), jnp.float32),
            jax.ShapeDtypeStruct((_G, 256), jnp.float32),
        ],
    )(batch2d, t_col, atom_table, nc_table, time_pad, w_h0, b_h0)


def _node_body(a_ref, b_ref, ta_ref, tg_ref, g_ref, beta_ref, out_ref):
    arow = a_ref[0]
    brow = b_ref[0]
    aio = lax.broadcasted_iota(jnp.int32, (128, _BN), 0)
    bio = lax.broadcasted_iota(jnp.int32, (_G, _BN), 0)
    oh_a = (arow == aio).astype(jnp.float32)
    oh_b = (brow == bio).astype(jnp.float32)
    dn = (((0,), (0,)), ((), ()))
    x = lax.dot_general(oh_a, ta_ref[...], dn,
                        preferred_element_type=jnp.float32)
    x = x + lax.dot_general(oh_b, tg_ref[...], dn,
                            preferred_element_type=jnp.float32)
    x = x * jax.nn.sigmoid(x)
    m = jnp.mean(x, axis=-1, keepdims=True)
    v = jnp.mean((x - m) ** 2, axis=-1, keepdims=True)
    out_ref[...] = (x - m) / jnp.sqrt(v + 1e-5) * g_ref[...] + beta_ref[...]


def _nodes(a3, b3, t_a, t_g, g_h0, beta_h0):
    return pl.pallas_call(
        _node_body,
        grid=(_NBLK,),
        in_specs=[
            pl.BlockSpec((1, 1, _BN), lambda i: (i, 0, 0)),
            pl.BlockSpec((1, 1, _BN), lambda i: (i, 0, 0)),
            pl.BlockSpec((128, 256), lambda i: (0, 0)),
            pl.BlockSpec((_G, 256), lambda i: (0, 0)),
            pl.BlockSpec((1, 256), lambda i: (0, 0)),
            pl.BlockSpec((1, 256), lambda i: (0, 0)),
        ],
        out_specs=pl.BlockSpec((_BN, 256), lambda i: (i, 0)),
        out_shape=jax.ShapeDtypeStruct((_N, 256), jnp.float32),
    )(a3, b3, t_a, t_g, g_h0, beta_h0)


def _edge_body(etab_hbm, e_hbm, out_hbm,
               shared, idx_all, rows0, rows1, rows2, rows3, trows,
               g0, g1, g2, g3, w0, w1, w2, w3, isem, tg, tw):
    rows = (rows0, rows1, rows2, rows3)
    gsem = (g0, g1, g2, g3)
    wsem = (w0, w1, w2, w3)

    sid = lax.axis_index("s")
    wid = sid * 2 + lax.axis_index("c")
    base = wid * _EPW

    @pl.when(sid == 0)
    def _():
        pltpu.sync_copy(etab_hbm, shared)
    plsc.subcore_barrier()

    pltpu.make_async_copy(e_hbm.at[pl.ds(base, _EPW)], idx_all, isem).start()
    pltpu.make_async_copy(e_hbm.at[pl.ds(base, _EPW)], idx_all, isem).wait()

    def g_start(c, b):
        pltpu.make_async_copy(shared.at[idx_all.at[pl.ds(c * _C, _C)]],
                              rows[b], gsem[b]).start()

    def g_wait(b):
        pltpu.make_async_copy(shared.at[idx_all.at[pl.ds(0, _C)]],
                              rows[b], gsem[b]).wait()

    def w_start(c, b):
        pltpu.make_async_copy(rows[b], out_hbm.at[pl.ds(base + c * _C, _C)],
                              wsem[b]).start()

    def w_wait(b):
        pltpu.make_async_copy(rows[b], out_hbm.at[pl.ds(0, _C)],
                              wsem[b]).wait()

    g_start(0, 0)
    g_start(1, 1)
    g_start(2, 2)
    pltpu.make_async_copy(
        shared.at[idx_all.at[pl.ds(_NFULL * _C, _TAIL)]], trows, tg).start()

    def body(i, _):
        for b in range(4):
            c = 4 * i + b
            g_wait(b)
            w_start(c, b)
            nb = (b + 3) % 4
            if b == 0:
                @pl.when(i > 0)
                def _():
                    w_wait(nb)
            else:
                w_wait(nb)
            g_start(c + 3, nb)
        return 0

    lax.fori_loop(0, (_NFULL // 4), body, 0)

    for c in (192, 193, 194):
        b = c % 4
        g_wait(b)
        w_start(c, b)
    pltpu.make_async_copy(
        shared.at[idx_all.at[pl.ds(0, _TAIL)]], trows, tg).wait()
    pltpu.make_async_copy(
        trows, out_hbm.at[pl.ds(base + _NFULL * _C, _TAIL)], tw).start()
    w_wait(3)
    w_wait(0)
    w_wait(1)
    w_wait(2)
    pltpu.make_async_copy(trows, out_hbm.at[pl.ds(0, _TAIL)], tw).wait()


def _edges(etab, e):
    mesh = plsc.VectorSubcoreMesh(core_axis_name="c", subcore_axis_name="s")
    fn = pl.kernel(
        _edge_body,
        out_type=jax.ShapeDtypeStruct((_E, 128), jnp.float32),
        mesh=mesh,
        scratch_types=[
            pltpu.VMEM_SHARED((8, 128), jnp.float32),
            pltpu.VMEM((_EPW,), jnp.int32),
            pltpu.VMEM((_C, 128), jnp.float32),
            pltpu.VMEM((_C, 128), jnp.float32),
            pltpu.VMEM((_C, 128), jnp.float32),
            pltpu.VMEM((_C, 128), jnp.float32),
            pltpu.VMEM((_TAIL, 128), jnp.float32),
            pltpu.SemaphoreType.DMA,
            pltpu.SemaphoreType.DMA,
            pltpu.SemaphoreType.DMA,
            pltpu.SemaphoreType.DMA,
            pltpu.SemaphoreType.DMA,
            pltpu.SemaphoreType.DMA,
            pltpu.SemaphoreType.DMA,
            pltpu.SemaphoreType.DMA,
            pltpu.SemaphoreType.DMA,
            pltpu.SemaphoreType.DMA,
            pltpu.SemaphoreType.DMA,
        ],
    )
    return fn(etab, e)


def kernel(a, e, edge_index, t, batch, atom_table, nc_table, time_table,
           edge_table, W_h0, b_h0, g_h0, beta_h0, W_e, b_e, g_e, beta_e):
    pad = _NP - _N
    a3 = jnp.pad(a, (0, pad)).reshape(_NBLK, 1, _BN)
    batch_p = jnp.pad(batch, (0, pad), constant_values=_G)
    b3 = batch_p.reshape(_NBLK, 1, _BN)
    batch2d = batch_p.reshape(_NBLK, _BN)
    t_col = t.reshape(_G, 1)
    time_pad = jnp.pad(time_table, ((0, 24), (0, 0)))

    etab = _prep_edge(edge_table, W_e, b_e.reshape(1, 128),
                      g_e.reshape(1, 128), beta_e.reshape(1, 128))
    t_a, t_g = _prep_node(batch2d, t_col, atom_table, nc_table, time_pad,
                          W_h0, b_h0.reshape(1, 256))

    e_embed = _edges(etab, e)
    h0 = _nodes(a3, b3, t_a, t_g,
                g_h0.reshape(1, 256), beta_h0.reshape(1, 256))
    return (h0, edge_index[0], edge_index[1], e_embed)


# speedup vs baseline: 1.0025x; 1.0025x over previous
import functools

import jax
import jax.numpy as jnp
from jax import lax
from jax.experimental import pallas as pl
from jax.experimental.pallas import tpu as pltpu
from jax.experimental.pallas import tpu_sc as plsc

_N = 50000
_E = 800000
_G = 256
_D = 64
_NP = 50176
_BN = 1024
_NBLK = _NP // _BN

_NW = 32
_EPW = _E // _NW
_C = 128
_NFULL = _EPW // _C
_TAIL = _EPW - _NFULL * _C
_R = 64


def _prep_edge_body(edge_ref, we_ref, be_ref, ge_ref, bee_ref, etab_ref):
    er = jnp.dot(edge_ref[...], we_ref[...],
                 preferred_element_type=jnp.float32) + be_ref[...]
    er = er * jax.nn.sigmoid(er)
    m = jnp.mean(er, axis=-1, keepdims=True)
    v = jnp.mean((er - m) ** 2, axis=-1, keepdims=True)
    etab_ref[...] = (er - m) / jnp.sqrt(v + 1e-5) * ge_ref[...] + bee_ref[...]


def _prep_edge(edge_table, w_e, b_e, g_e, beta_e):
    return pl.pallas_call(
        _prep_edge_body,
        out_shape=jax.ShapeDtypeStruct((8, 128), jnp.float32),
    )(edge_table, w_e, b_e, g_e, beta_e)


def _prep_node_body(batch_ref, t_ref, atom_ref, nc_ref, time_ref,
                    wh_ref, bh_ref, ta_ref, tg_ref):
    gio = lax.broadcasted_iota(jnp.int32, (_G, _BN), 0)

    def step(i, acc):
        row = batch_ref[pl.ds(i, 1), :]
        cmp = (row == gio).astype(jnp.float32)
        return acc + jnp.sum(cmp, axis=1, keepdims=True)

    counts = lax.fori_loop(0, _NBLK, step,
                           jnp.zeros((_G, 1), jnp.float32))
    counts = jnp.clip(counts.astype(jnp.int32), 0, 1023)

    vio = lax.broadcasted_iota(jnp.int32, (_G, 1024), 1)
    nc_oh = (counts == vio).astype(jnp.float32)
    nc_g = jnp.dot(nc_oh, nc_ref[...],
                   preferred_element_type=jnp.float32)
    t_oh = (t_ref[...] == vio).astype(jnp.float32)
    t_g = jnp.dot(t_oh, time_ref[...],
                  preferred_element_type=jnp.float32)

    wh = wh_ref[...]
    tg_ref[...] = (
        jnp.dot(nc_g, wh[64:128, :], preferred_element_type=jnp.float32)
        + jnp.dot(t_g, wh[128:192, :], preferred_element_type=jnp.float32)
        + bh_ref[...])
    ta_ref[...] = jnp.dot(atom_ref[...], wh[0:64, :],
                          preferred_element_type=jnp.float32)


def _prep_node(batch2d, t_col, atom_table, nc_table, time_pad, w_h0, b_h0):
    return pl.pallas_call(
        _prep_node_body,
        out_shape=[
            jax.ShapeDtypeStruct((128, 256), jnp.float32),
            jax.ShapeDtypeStruct((_G, 256), jnp.float32),
        ],
    )(batch2d, t_col, atom_table, nc_table, time_pad, w_h0, b_h0)


def _node_body(a_ref, b_ref, ta_ref, tg_ref, g_ref, beta_ref, out_ref):
    arow = a_ref[0]
    brow = b_ref[0]
    aio = lax.broadcasted_iota(jnp.int32, (128, _BN), 0)
    bio = lax.broadcasted_iota(jnp.int32, (_G, _BN), 0)
    oh_a = (arow == aio).astype(jnp.float32)
    oh_b = (brow == bio).astype(jnp.float32)
    dn = (((0,), (0,)), ((), ()))
    x = lax.dot_general(oh_a, ta_ref[...], dn,
                        preferred_element_type=jnp.float32)
    x = x + lax.dot_general(oh_b, tg_ref[...], dn,
                            preferred_element_type=jnp.float32)
    x = x * jax.nn.sigmoid(x)
    m = jnp.mean(x, axis=-1, keepdims=True)
    v = jnp.mean((x - m) ** 2, axis=-1, keepdims=True)
    out_ref[...] = (x - m) / jnp.sqrt(v + 1e-5) * g_ref[...] + beta_ref[...]


def _nodes(a3, b3, t_a, t_g, g_h0, beta_h0):
    return pl.pallas_call(
        _node_body,
        grid=(_NBLK,),
        in_specs=[
            pl.BlockSpec((1, 1, _BN), lambda i: (i, 0, 0)),
            pl.BlockSpec((1, 1, _BN), lambda i: (i, 0, 0)),
            pl.BlockSpec((128, 256), lambda i: (0, 0)),
            pl.BlockSpec((_G, 256), lambda i: (0, 0)),
            pl.BlockSpec((1, 256), lambda i: (0, 0)),
            pl.BlockSpec((1, 256), lambda i: (0, 0)),
        ],
        out_specs=pl.BlockSpec((_BN, 256), lambda i: (i, 0)),
        out_shape=jax.ShapeDtypeStruct((_N, 256), jnp.float32),
    )(a3, b3, t_a, t_g, g_h0, beta_h0)


def _edge_body(etab_hbm, e_hbm, out_hbm,
               shared, idx_all, rows0, rows1, rows2, rows3,
               g0, g1, g2, g3, w0, w1, w2, w3, isem):
    rows = (rows0, rows1, rows2, rows3)
    gsem = (g0, g1, g2, g3)
    wsem = (w0, w1, w2, w3)

    sid = lax.axis_index("s")
    wid = sid * 2 + lax.axis_index("c")
    base = wid * _EPW

    @pl.when(sid == 0)
    def _():
        pltpu.sync_copy(etab_hbm, shared)
    plsc.subcore_barrier()

    pltpu.make_async_copy(e_hbm.at[pl.ds(base, _EPW)], idx_all, isem).start()
    pltpu.make_async_copy(e_hbm.at[pl.ds(base, _EPW)], idx_all, isem).wait()

    def g_start(c, b):
        pltpu.make_async_copy(shared.at[idx_all.at[pl.ds(c * _C, _C)]],
                              rows[b], gsem[b]).start()

    def g_wait(b):
        pltpu.make_async_copy(shared.at[idx_all.at[pl.ds(0, _C)]],
                              rows[b], gsem[b]).wait()

    def w_start(c, b):
        pltpu.make_async_copy(rows[b], out_hbm.at[pl.ds(base + c * _C, _C)],
                              wsem[b]).start()

    def w_wait(b):
        pltpu.make_async_copy(rows[b], out_hbm.at[pl.ds(0, _C)],
                              wsem[b]).wait()

    g_start(0, 0)
    g_start(1, 1)
    g_start(2, 2)

    def body(i, _):
        for b in range(4):
            c = 4 * i + b
            g_wait(b)
            w_start(c, b)
            nb = (b + 3) % 4
            if b == 0:
                @pl.when(i > 0)
                def _():
                    w_wait(nb)
            else:
                w_wait(nb)
            g_start(c + 3, nb)
        return 0

    lax.fori_loop(0, (_NFULL // 4), body, 0)

    for c in (192, 193, 194):
        b = c % 4
        g_wait(b)
        w_start(c, b)
    w_wait(3)
    w_wait(0)
    w_wait(1)
    w_wait(2)

    pltpu.make_async_copy(
        shared.at[idx_all.at[pl.ds(_NFULL * _C, _TAIL)]],
        rows0.at[pl.ds(0, _TAIL)], g0).start()
    pltpu.make_async_copy(
        shared.at[idx_all.at[pl.ds(0, _TAIL)]],
        rows0.at[pl.ds(0, _TAIL)], g0).wait()
    pltpu.make_async_copy(
        rows0.at[pl.ds(0, _TAIL)],
        out_hbm.at[pl.ds(base + _NFULL * _C, _TAIL)], w0).start()
    pltpu.make_async_copy(rows0.at[pl.ds(0, _TAIL)],
                          out_hbm.at[pl.ds(0, _TAIL)], w0).wait()


def _edges(etab, e):
    mesh = plsc.VectorSubcoreMesh(core_axis_name="c", subcore_axis_name="s")
    fn = pl.kernel(
        _edge_body,
        out_type=jax.ShapeDtypeStruct((_E, 128), jnp.float32),
        mesh=mesh,
        scratch_types=[
            pltpu.VMEM_SHARED((8, 128), jnp.float32),
            pltpu.VMEM((_EPW,), jnp.int32),
            pltpu.VMEM((_C, 128), jnp.float32),
            pltpu.VMEM((_C, 128), jnp.float32),
            pltpu.VMEM((_C, 128), jnp.float32),
            pltpu.VMEM((_C, 128), jnp.float32),
            pltpu.SemaphoreType.DMA,
            pltpu.SemaphoreType.DMA,
            pltpu.SemaphoreType.DMA,
            pltpu.SemaphoreType.DMA,
            pltpu.SemaphoreType.DMA,
            pltpu.SemaphoreType.DMA,
            pltpu.SemaphoreType.DMA,
            pltpu.SemaphoreType.DMA,
            pltpu.SemaphoreType.DMA,
        ],
    )
    return fn(etab, e)


def kernel(a, e, edge_index, t, batch, atom_table, nc_table, time_table,
           edge_table, W_h0, b_h0, g_h0, beta_h0, W_e, b_e, g_e, beta_e):
    pad = _NP - _N
    a3 = jnp.pad(a, (0, pad)).reshape(_NBLK, 1, _BN)
    batch_p = jnp.pad(batch, (0, pad), constant_values=_G)
    b3 = batch_p.reshape(_NBLK, 1, _BN)
    batch2d = batch_p.reshape(_NBLK, _BN)
    t_col = t.reshape(_G, 1)
    time_pad = jnp.pad(time_table, ((0, 24), (0, 0)))

    etab = _prep_edge(edge_table, W_e, b_e.reshape(1, 128),
                      g_e.reshape(1, 128), beta_e.reshape(1, 128))
    t_a, t_g = _prep_node(batch2d, t_col, atom_table, nc_table, time_pad,
                          W_h0, b_h0.reshape(1, 256))

    e_embed = _edges(etab, e)
    h0 = _nodes(a3, b3, t_a, t_g,
                g_h0.reshape(1, 256), beta_h0.reshape(1, 256))
    return (h0, edge_index[0], edge_index[1], e_embed)
